# Initial kernel scaffold; baseline (speedup 1.0000x reference)
import jax
import jax.numpy as jnp
from jax.experimental import pallas as pl


def kernel(x, y, Wenc, benc, Wq, bq, Wk, bk, Wv, bv, Wo, bo, W1, b1, W2, b2, ln1g, ln1b, ln2g, ln2b, ae_e1, ae_e1b, ae_e2, ae_e2b, ae_d1, ae_d1b, ae_d2, ae_d2b):
    raise NotImplementedError("write your pallas kernel here")


# speedup vs baseline: 8.2168x; 8.2168x over previous
import jax
import jax.numpy as jnp
from jax import lax
from jax.experimental import pallas as pl
from jax.experimental.pallas import tpu as pltpu

B = 32; S = 256; IN = 128; D = 128; H = 8; L = 2; NC = 16; V = 16; FF = 512
DH = D // H
TS = 8


def _layernorm(x, g, b):
    m = jnp.mean(x, axis=-1, keepdims=True)
    v = jnp.mean((x - m) ** 2, axis=-1, keepdims=True)
    return (x - m) / jnp.sqrt(v + 1e-5) * g + b


def _mm(a, b):
    return jnp.dot(a.astype(jnp.bfloat16), b.astype(jnp.bfloat16),
                   preferred_element_type=jnp.float32)


def _transformer_kernel(x_ref, Wenc_ref, benc_ref, Wq_ref, bq_ref, Wk_ref,
                        bk_ref, Wv_ref, bv_ref, Wo_ref, bo_ref, W1_ref, b1_ref,
                        W2_ref, b2_ref, ln1g_ref, ln1b_ref, ln2g_ref, ln2b_ref,
                        e1_ref, e1b_ref, e2_ref, e2b_ref, d1_ref, d1b_ref,
                        d2_ref, d2b_ref, out_ref):
    xb = x_ref[0]
    h = _mm(xb, Wenc_ref[...]) + benc_ref[...]
    r = lax.broadcasted_iota(jnp.int32, (H * S, D), 0)
    c = lax.broadcasted_iota(jnp.int32, (H * S, D), 1)
    bd_mask = (r // S) == (c // DH)
    for l in range(L):
        q = _mm(h, Wq_ref[l]) + bq_ref[l]
        k = _mm(h, Wk_ref[l]) + bk_ref[l]
        v = _mm(h, Wv_ref[l]) + bv_ref[l]
        kbd = jnp.where(bd_mask,
                        jnp.concatenate([k] * H, axis=0).astype(jnp.bfloat16),
                        jnp.zeros((), jnp.bfloat16))
        scoresT = lax.dot_general(kbd, q.astype(jnp.bfloat16),
                                  (((1,), (1,)), ((), ())),
                                  preferred_element_type=jnp.float32) / 4.0
        o_parts = []
        for hh in range(H):
            shT = scoresT[hh * S:(hh + 1) * S, :]
            mT = jnp.max(shT, axis=0, keepdims=True)
            eT = jnp.exp(shT - mT)
            pT = eT / jnp.sum(eT, axis=0, keepdims=True)
            ph = pT.T
            vh = v[:, hh * DH:(hh + 1) * DH]
            o_parts.append(
                lax.dot_general(ph.astype(jnp.bfloat16),
                                vh.astype(jnp.bfloat16),
                                (((1,), (0,)), ((), ())),
                                preferred_element_type=jnp.float32))
        o = jnp.concatenate(o_parts, axis=1)
        o = _mm(o, Wo_ref[l]) + bo_ref[l]
        h = _layernorm(h + o, ln1g_ref[l], ln1b_ref[l])
        f = jnp.maximum(_mm(h, W1_ref[l]) + b1_ref[l], 0.0)
        f = _mm(f, W2_ref[l]) + b2_ref[l]
        h = _layernorm(h + f, ln2g_ref[l], ln2b_ref[l])
    e_ = jnp.maximum(_mm(h, e1_ref[...]) + e1b_ref[...], 0.0)
    e_ = _mm(e_, e2_ref[...]) + e2b_ref[...]
    t = jnp.maximum(_mm(e_, d1_ref[...]) + d1b_ref[...], 0.0)
    out_ref[0] = _mm(t, d2_ref[...]) + d2b_ref[...]


def _assign_kernel(itc_ref, ylab_ref, ytrue_ref, assigned_ref, tot_ref,
                   ari_ref, cont_ref):
    i = pl.program_id(0)
    nsteps = S // TS
    dt_list = []
    for sl in range(TS):
        a = itc_ref[sl]
        diff = a[:, None, :] - a[None, :, :]
        dt_list.append(jnp.sum(diff * diff, axis=-1))
    DT = jnp.concatenate(dt_list, axis=1)
    E = jnp.exp(-DT)
    ss_list = []
    for sl in range(TS):
        e_sl = E[:, sl * B:(sl + 1) * B]
        ss_list.append(jnp.sum(e_sl, axis=-1, keepdims=True).T)
    SST = jnp.concatenate(ss_list, axis=1)
    PT = E / SST
    iota_c = lax.broadcasted_iota(jnp.int32, (B, TS * B), 0)
    one = jnp.ones((), jnp.int32)
    zero = jnp.zeros((), jnp.int32)
    rankT = jnp.zeros((B, TS * B), jnp.int32)
    for cp in range(B):
        rowv = PT[cp:cp + 1, :]
        gt = jnp.where(rowv > PT, one, zero)
        eq = jnp.where((rowv == PT) & (iota_c > cp), one, zero)
        rankT = rankT + gt + eq
    T16T = (rankT < NC).astype(jnp.float32)
    mcT = jnp.sum(jnp.where((rankT == iota_c) & (iota_c < NC), 1.0, 0.0),
                  axis=0, keepdims=True)
    iota_vN = lax.broadcasted_iota(jnp.int32, (V, B), 0)
    contrib = jnp.zeros((1, 1), jnp.float32)
    cntT_list = []
    for sl in range(TS):
        rsT = jnp.sum(DT[:, sl * B:(sl + 1) * B], axis=0, keepdims=True)
        contrib = contrib + jnp.sum(mcT[:, sl * B:(sl + 1) * B] * rsT,
                                    keepdims=True)
        ohT = (ylab_ref[sl:sl + 1, :] == iota_vN).astype(jnp.float32)
        t16_sl = T16T[:, sl * B:(sl + 1) * B]
        cntT_list.append(
            lax.dot_general(ohT.astype(jnp.bfloat16),
                            t16_sl.astype(jnp.bfloat16),
                            (((1,), (0,)), ((), ())),
                            preferred_element_type=jnp.float32))
    CT = jnp.concatenate(cntT_list, axis=1)
    best = CT[0:1, :]
    bi = jnp.zeros((1, TS * B), jnp.int32)
    for v in range(1, V):
        cv = CT[v:v + 1, :]
        upd = cv > best
        bi = jnp.where(upd, v, bi)
        best = jnp.where(upd, cv, best)
    assigned_ref[0] = bi
    iota_vL = lax.broadcasted_iota(jnp.int32, (V, TS * B), 0)
    pa = (bi == iota_vL).astype(jnp.float32)
    tb = (ytrue_ref[0] == iota_vL).astype(jnp.float32)
    cstep = lax.dot_general(pa, tb, (((1,), (1,)), ((), ())),
                            preferred_element_type=jnp.float32)

    @pl.when(i == 0)
    def _():
        cont_ref[...] = cstep
        tot_ref[...] = contrib

    @pl.when(i > 0)
    def _():
        cont_ref[...] = cont_ref[...] + cstep
        tot_ref[...] = tot_ref[...] + contrib

    @pl.when(i == nsteps - 1)
    def _():
        cont = cont_ref[...]
        c2 = lambda mm: mm * (mm - 1.0) / 2.0
        sumc = jnp.sum(c2(cont), keepdims=True)
        rows = jnp.sum(cont, axis=1, keepdims=True)
        cols = jnp.sum(cont, axis=0, keepdims=True)
        a_ = jnp.sum(c2(rows), keepdims=True)
        b_ = jnp.sum(c2(cols), keepdims=True)
        tot_pairs = 33550336.0
        expv = a_ * b_ / tot_pairs
        maxi = (a_ + b_) / 2.0
        ari_ref[...] = (sumc - expv) / (maxi - expv + 1e-12)


def kernel(x, y, Wenc, benc, Wq, bq, Wk, bk, Wv, bv, Wo, bo, W1, b1, W2, b2,
           ln1g, ln1b, ln2g, ln2b, ae_e1, ae_e1b, ae_e2, ae_e2b, ae_d1,
           ae_d1b, ae_d2, ae_d2b):
    f32 = jnp.float32
    wfull = lambda a: pl.BlockSpec(a.shape, lambda b: (0,) * a.ndim)
    itc = pl.pallas_call(
        _transformer_kernel,
        grid=(B,),
        in_specs=[pl.BlockSpec((1, S, IN), lambda b: (b, 0, 0))] + [
            wfull(a) for a in (Wenc, benc, Wq, bq, Wk, bk, Wv, bv, Wo, bo, W1,
                               b1, W2, b2, ln1g, ln1b, ln2g, ln2b, ae_e1,
                               ae_e1b, ae_e2, ae_e2b, ae_d1, ae_d1b, ae_d2,
                               ae_d2b)],
        out_specs=pl.BlockSpec((1, S, D), lambda b: (b, 0, 0)),
        out_shape=jax.ShapeDtypeStruct((B, S, D), f32),
        compiler_params=pltpu.CompilerParams(
            dimension_semantics=("arbitrary",)),
    )(x, Wenc, benc, Wq, bq, Wk, bk, Wv, bv, Wo, bo, W1, b1, W2, b2, ln1g,
      ln1b, ln2g, ln2b, ae_e1, ae_e1b, ae_e2, ae_e2b, ae_d1, ae_d1b, ae_d2,
      ae_d2b)

    itc_t = itc.transpose(1, 0, 2)
    ylab = y[:, :, 0].T.astype(jnp.int32)
    ytrue = y.reshape(S // TS, 1, TS * B).astype(jnp.int32)
    assigned_flat, tot, ari = pl.pallas_call(
        _assign_kernel,
        grid=(S // TS,),
        in_specs=[
            pl.BlockSpec((TS, B, D), lambda i: (i, 0, 0)),
            pl.BlockSpec((TS, B), lambda i: (i, 0)),
            pl.BlockSpec((1, 1, TS * B), lambda i: (i, 0, 0)),
        ],
        out_specs=[
            pl.BlockSpec((1, 1, TS * B), lambda i: (i, 0, 0)),
            pl.BlockSpec((1, 1), lambda i: (0, 0)),
            pl.BlockSpec((1, 1), lambda i: (0, 0)),
        ],
        out_shape=[
            jax.ShapeDtypeStruct((S // TS, 1, TS * B), jnp.int32),
            jax.ShapeDtypeStruct((1, 1), f32),
            jax.ShapeDtypeStruct((1, 1), f32),
        ],
        scratch_shapes=[pltpu.VMEM((V, V), f32)],
        compiler_params=pltpu.CompilerParams(
            dimension_semantics=("arbitrary",)),
    )(itc_t, ylab, ytrue)
    assigned = assigned_flat.reshape(S, B)
    return tot[0, 0], ari[0, 0], assigned, itc
